# baseline (speedup 1.0000x reference)
import dataclasses
import functools

import jax
import jax.numpy as jnp
from jax import lax
from jax.experimental import pallas as pl
from jax.experimental.pallas import tpu as pltpu
from jax.experimental.pallas import tpu_sc as plsc

_N = 10000
_E = 320000
_D = 128
_ALPHA = 0.1
_EPS = 1e-5

_NC = 2
_NS = 16
_L = 16
_NW = _NC * _NS

_CH = 128
_CB = 8
_NBLK = 10
_NCHT = _CB * _NBLK
_EPAD = _NW * _NCHT * _CH
_NPAD = 10240
_RPT = _NPAD // _NS
_CSEG = 160


def _rsqrt16(d):
    i = plsc.bitcast(d, jnp.int32)
    i = jnp.int32(0x5F3759DF) - lax.shift_right_logical(i, 1)
    y = plsc.bitcast(i, jnp.float32)
    for _ in range(3):
        y = y * (1.5 - 0.5 * d * y * y)
    return y


def _sc_gcn_agg(x, row3, col3, ew3):
    mesh = plsc.VectorSubcoreMesh(core_axis_name="c", subcore_axis_name="s")
    cp = pltpu.CompilerParams()
    if "needs_layout_passes" in pltpu.CompilerParams.__dataclass_fields__:
        cp = dataclasses.replace(cp, needs_layout_passes=False)
    if "use_tc_tiling_on_sc" in pltpu.CompilerParams.__dataclass_fields__:
        cp = dataclasses.replace(cp, use_tc_tiling_on_sc=False)

    @functools.partial(
        pl.kernel,
        compiler_params=cp,
        out_type=(
            jax.ShapeDtypeStruct((_NC, _NPAD, _D), jnp.float32),
            jax.ShapeDtypeStruct((_NPAD,), jnp.float32),
        ),
        mesh=mesh,
        scratch_types=[
            pltpu.VMEM((_CB, _CH), jnp.int32),
            pltpu.VMEM((_CB, _CH), jnp.int32),
            pltpu.VMEM((_CB, _CH), jnp.float32),
            pltpu.VMEM((_NPAD,), jnp.float32),
            pltpu.VMEM((_CSEG, _L), jnp.float32),
            pltpu.VMEM((_CH, _L), jnp.float32),
            pltpu.VMEM((_CH, _D), jnp.float32),
            pltpu.VMEM((_CH,), jnp.float32),
            pltpu.VMEM_SHARED((_NPAD, _L), jnp.float32),
            pltpu.VMEM_SHARED((_NPAD, _D), jnp.float32),
        ],
    )
    def k(x_hbm, row_hbm, col_hbm, ew_hbm, agg_out, dinv_out,
          rowbuf, colbuf, ewbuf, dinvv, cbuf, valbuf, msgbuf, normbuf,
          degmat, aggsh):
        c = lax.axis_index("c")
        s = lax.axis_index("s")
        iota16 = lax.iota(jnp.int32, _L)
        zero16i = jnp.zeros((_L,), jnp.int32)
        z16 = jnp.zeros((_L,), jnp.float32)

        with jax.named_scope("ph_zero"):
            @pl.loop(0, _CH)
            def _(e):
                valbuf[e, :] = z16
                for g in range(_D // _L):
                    msgbuf[e, pl.ds(g * _L, _L)] = z16

            for i in range(_RPT // _CH):
                pltpu.sync_copy(valbuf,
                                degmat.at[pl.ds(s * _RPT + i * _CH, _CH)])
                pltpu.sync_copy(msgbuf,
                                aggsh.at[pl.ds(s * _RPT + i * _CH, _CH)])

            plsc.subcore_barrier()

        with jax.named_scope("ph_deg"):
            for h in range(_NC):
                slab = h * _NS + s

                @pl.loop(0, _NBLK)
                def _(jb, slab=slab):
                    pltpu.sync_copy(col_hbm.at[slab, pl.ds(jb * _CB, _CB)],
                                    colbuf)
                    pltpu.sync_copy(ew_hbm.at[slab, pl.ds(jb * _CB, _CB)],
                                    ewbuf)
                    for j8 in range(_CB):
                        for g in range(_CH // _L):
                            ew16 = ewbuf[j8, pl.ds(g * _L, _L)]
                            plsc.store_scatter(valbuf,
                                               [g * _L + iota16, zero16i],
                                               ew16)
                        pltpu.sync_copy(valbuf, degmat.at[colbuf.at[j8]],
                                        add=True)

            plsc.subcore_barrier()

        with jax.named_scope("ph_rsqrt"):
            @pl.loop(0, _NPAD // _CSEG)
            def _(seg):
                pltpu.sync_copy(degmat.at[pl.ds(seg * _CSEG, _CSEG)], cbuf)

                @pl.loop(0, _CSEG // _L)
                def _(g, seg=seg):
                    r16 = g * _L + iota16
                    d16 = plsc.load_gather(cbuf, [r16, zero16i])
                    dinvv[pl.ds(seg * _CSEG + g * _L, _L)] = _rsqrt16(d16 + 1.0)

        aslab = c * _NS + s

        with jax.named_scope("ph_agg"):
            @pl.loop(0, _NBLK)
            def _(jb):
                pltpu.sync_copy(row_hbm.at[aslab, pl.ds(jb * _CB, _CB)],
                                rowbuf)
                pltpu.sync_copy(col_hbm.at[aslab, pl.ds(jb * _CB, _CB)],
                                colbuf)
                pltpu.sync_copy(ew_hbm.at[aslab, pl.ds(jb * _CB, _CB)],
                                ewbuf)
                for j8 in range(_CB):
                    pltpu.sync_copy(x_hbm.at[rowbuf.at[j8]], msgbuf)
                    for g in range(_CH // _L):
                        sl = pl.ds(g * _L, _L)
                        r16 = rowbuf[j8, sl]
                        c16 = colbuf[j8, sl]
                        ew16 = ewbuf[j8, sl]
                        dr = plsc.load_gather(dinvv, [r16])
                        dc = plsc.load_gather(dinvv, [c16])
                        normbuf[sl] = dr * ew16 * dc

                    @pl.loop(0, _CH // _L)
                    def _(eo, j8=j8):
                        n16 = normbuf[pl.ds(eo * _L, _L)]
                        for kk in range(_L):
                            ne = n16[kk]
                            for g in range(_D // _L):
                                sl = pl.ds(g * _L, _L)
                                msgbuf[eo * _L + kk, sl] = \
                                    msgbuf[eo * _L + kk, sl] * ne

                    pltpu.sync_copy(msgbuf, aggsh.at[colbuf.at[j8]], add=True)

        plsc.subcore_barrier()

        pltpu.sync_copy(aggsh.at[pl.ds(s * _RPT, _RPT)],
                        agg_out.at[c, pl.ds(s * _RPT, _RPT)])

        @pl.when(c == 0)
        def _():
            pltpu.sync_copy(dinvv.at[pl.ds(s * _RPT, _RPT)],
                            dinv_out.at[pl.ds(s * _RPT, _RPT)])

    return k(x, row3, col3, ew3)


def _tc_tail(agg_ref, x_ref, x0_ref, dinv_ref, w_ref, g_ref, b_ref, y_ref):
    dsq = dinv_ref[...] * dinv_ref[...]
    agg = agg_ref[0] + agg_ref[1] + x_ref[...] * dsq
    h = (1.0 - _ALPHA) * agg + _ALPHA * x0_ref[...]
    out = jnp.dot(h, w_ref[...], preferred_element_type=jnp.float32,
                  precision=lax.Precision.HIGHEST)
    out = jnp.maximum(out, 0.0)
    mean = jnp.sum(out, axis=0) / _N
    msq = jnp.sum(out * out, axis=0) / _N
    var = msq - mean * mean
    scale = g_ref[...] * lax.rsqrt(var + _EPS)[None, :]
    y_ref[...] = (out - mean[None, :]) * scale + b_ref[...]


def kernel(x, x_orig, edge_index, edge_weight, W, gamma, beta):
    row = edge_index[0]
    col = edge_index[1]
    pad = _EPAD - _E
    padidx = jnp.arange(pad, dtype=jnp.int32) % _N
    zpad = jnp.zeros((pad,), dtype=jnp.float32)
    row3 = jnp.concatenate([row, padidx]).reshape(_NW, _NCHT, _CH)
    col3 = jnp.concatenate([col, padidx]).reshape(_NW, _NCHT, _CH)
    ew3 = jnp.concatenate([edge_weight, zpad]).reshape(_NW, _NCHT, _CH)

    aggp, dinv = _sc_gcn_agg(x, row3, col3, ew3)

    x_pad = jnp.pad(x, ((0, _NPAD - _N), (0, 0)))
    x0_pad = jnp.pad(x_orig, ((0, _NPAD - _N), (0, 0)))
    y_full = pl.pallas_call(
        _tc_tail,
        out_shape=jax.ShapeDtypeStruct((_NPAD, _D), jnp.float32),
    )(aggp, x_pad, x0_pad, dinv[:, None], W, gamma[None, :], beta[None, :])

    y = y_full[:_N]
    return (y, x_orig, edge_index, edge_weight, x)


# speedup vs baseline: 1.0119x; 1.0119x over previous
import dataclasses
import functools

import jax
import jax.numpy as jnp
from jax import lax
from jax.experimental import pallas as pl
from jax.experimental.pallas import tpu as pltpu
from jax.experimental.pallas import tpu_sc as plsc

_N = 10000
_E = 320000
_D = 128
_ALPHA = 0.1
_EPS = 1e-5

_NC = 2
_NS = 16
_L = 16
_NW = _NC * _NS
_DH = _D // _NC

_CH = 128
_CB = 8
_NBLK = 10
_NCHT = _CB * _NBLK
_EPAD = _NW * _NCHT * _CH
_NPAD = 10240
_RPT = _NPAD // _NS
_CSEG = 160
_NMB = 4


def _rsqrt16(d):
    i = plsc.bitcast(d, jnp.int32)
    i = jnp.int32(0x5F3759DF) - lax.shift_right_logical(i, 1)
    y = plsc.bitcast(i, jnp.float32)
    for _ in range(3):
        y = y * (1.5 - 0.5 * d * y * y)
    return y


def _sc_gcn_agg(x2, idx4):
    mesh = plsc.VectorSubcoreMesh(core_axis_name="c", subcore_axis_name="s")
    cp = pltpu.CompilerParams()
    if "needs_layout_passes" in pltpu.CompilerParams.__dataclass_fields__:
        cp = dataclasses.replace(cp, needs_layout_passes=False)
    if "use_tc_tiling_on_sc" in pltpu.CompilerParams.__dataclass_fields__:
        cp = dataclasses.replace(cp, use_tc_tiling_on_sc=False)

    @functools.partial(
        pl.kernel,
        compiler_params=cp,
        out_type=(
            jax.ShapeDtypeStruct((_NC, _NPAD, _DH), jnp.float32),
            jax.ShapeDtypeStruct((_NPAD,), jnp.float32),
        ),
        mesh=mesh,
        scratch_types=[
            pltpu.VMEM((3, _CB, _CH), jnp.int32),
            pltpu.VMEM((_NMB, _CH), jnp.int32),
            pltpu.VMEM((_NPAD,), jnp.float32),
            pltpu.VMEM((_CSEG, _L), jnp.float32),
            pltpu.VMEM((2, _CH, _L), jnp.float32),
            pltpu.VMEM((_NMB, _CH, _DH), jnp.float32),
            pltpu.VMEM((_CH,), jnp.float32),
            pltpu.VMEM_SHARED((_NPAD, _L), jnp.float32),
            pltpu.VMEM_SHARED((_NPAD, _DH), jnp.float32),
            pltpu.VMEM_SHARED((_NPAD,), jnp.float32),
        ] + [pltpu.SemaphoreType.DMA] * (2 * _NMB + 2),
    )
    def k(x_hbm, idx_hbm, agg_out, dinv_out,
          idxbuf, gidxbuf, dinvv, cbuf, valbuf, msgbuf, normbuf,
          degmat, aggsh, dinvsh,
          sg0, sg1, sg2, sg3, ss0, ss1, ss2, ss3, sd0, sd1):
        semg = [sg0, sg1, sg2, sg3]
        sems = [ss0, ss1, ss2, ss3]
        semd = [sd0, sd1]
        c = lax.axis_index("c")
        s = lax.axis_index("s")
        iota16 = lax.iota(jnp.int32, _L)
        zero16i = jnp.zeros((_L,), jnp.int32)
        z16 = jnp.zeros((_L,), jnp.float32)

        with jax.named_scope("ph_zero"):
            @pl.loop(0, _CH)
            def _(e):
                valbuf[0, e, :] = z16
                valbuf[1, e, :] = z16
                for g in range(_DH // _L):
                    msgbuf[0, e, pl.ds(g * _L, _L)] = z16

            for i in range(_RPT // _CH):
                base = s * _RPT + i * _CH
                pltpu.sync_copy(valbuf.at[0], degmat.at[pl.ds(base, _CH)])
                pltpu.sync_copy(msgbuf.at[0], aggsh.at[pl.ds(base, _CH)])

            plsc.subcore_barrier()

        with jax.named_scope("ph_deg"):
            for h in range(_NC):
                slab = h * _NS + s

                @pl.loop(0, _NBLK)
                def _(jb, slab=slab):
                    pltpu.sync_copy(idx_hbm.at[slab, jb], idxbuf)
                    hs = {}
                    for j8 in range(_CB):
                        b = j8 % 2
                        if j8 >= 2:
                            hs[b].wait()
                        for g in range(_CH // _L):
                            ew16 = plsc.bitcast(
                                idxbuf[2, j8, pl.ds(g * _L, _L)], jnp.float32)
                            plsc.store_scatter(
                                valbuf.at[b], [g * _L + iota16, zero16i], ew16)
                        hs[b] = pltpu.async_copy(
                            valbuf.at[b], degmat.at[idxbuf.at[1, j8]],
                            semd[b], add=True)
                    hs[0].wait()
                    hs[1].wait()

            plsc.subcore_barrier()

        with jax.named_scope("ph_rsqrt"):
            for seg in range(_RPT // _CSEG):
                base = s * _RPT + seg * _CSEG
                pltpu.sync_copy(degmat.at[pl.ds(base, _CSEG)], cbuf)

                @pl.loop(0, _CSEG // _L)
                def _(g, base=base):
                    r16 = g * _L + iota16
                    d16 = plsc.load_gather(cbuf, [r16, zero16i])
                    dinvv[pl.ds(base + g * _L, _L)] = _rsqrt16(d16 + 1.0)

            pltpu.sync_copy(dinvv.at[pl.ds(s * _RPT, _RPT)],
                            dinvsh.at[pl.ds(s * _RPT, _RPT)])
            plsc.subcore_barrier()
            pltpu.sync_copy(dinvsh, dinvv)

        with jax.named_scope("ph_agg"):
            def build_gidx(q, b):
                for g in range(_CH // _L):
                    sl = pl.ds(g * _L, _L)
                    gidxbuf[b, sl] = idxbuf[0, q, sl] * 2 + c

            for h in range(_NC):
                slab = h * _NS + s

                @pl.loop(0, _NBLK)
                def _(jb, slab=slab):
                    pltpu.sync_copy(idx_hbm.at[slab, jb], idxbuf)
                    hg, hsc = {}, {}
                    for q in range(2):
                        build_gidx(q, q)
                        hg[q] = pltpu.async_copy(
                            x_hbm.at[gidxbuf.at[q]], msgbuf.at[q], semg[q])
                    for j8 in range(_CB):
                        b = j8 % _NMB
                        hg[b].wait()
                        if j8 + 2 < _CB:
                            b2 = (j8 + 2) % _NMB
                            if j8 >= 2:
                                hsc[b2].wait()
                            build_gidx(j8 + 2, b2)
                            hg[b2] = pltpu.async_copy(
                                x_hbm.at[gidxbuf.at[b2]], msgbuf.at[b2],
                                semg[b2])
                        for g in range(_CH // _L):
                            sl = pl.ds(g * _L, _L)
                            r16 = idxbuf[0, j8, sl]
                            c16 = idxbuf[1, j8, sl]
                            ew16 = plsc.bitcast(idxbuf[2, j8, sl],
                                                jnp.float32)
                            dr = plsc.load_gather(dinvv, [r16])
                            dc = plsc.load_gather(dinvv, [c16])
                            normbuf[sl] = dr * ew16 * dc

                        @pl.loop(0, _CH // _L)
                        def _(eo, b=b):
                            n16 = normbuf[pl.ds(eo * _L, _L)]
                            for kk in range(_L):
                                ne = n16[kk]
                                for g in range(_DH // _L):
                                    sl = pl.ds(g * _L, _L)
                                    msgbuf[b, eo * _L + kk, sl] = \
                                        msgbuf[b, eo * _L + kk, sl] * ne

                        hsc[b] = pltpu.async_copy(
                            msgbuf.at[b], aggsh.at[idxbuf.at[1, j8]],
                            sems[b], add=True)
                    for b in range(_NMB):
                        hsc[b].wait()

            plsc.subcore_barrier()

        with jax.named_scope("ph_out"):
            pltpu.sync_copy(aggsh.at[pl.ds(s * _RPT, _RPT)],
                            agg_out.at[c, pl.ds(s * _RPT, _RPT)])

            @pl.when(c == 0)
            def _():
                pltpu.sync_copy(dinvv.at[pl.ds(s * _RPT, _RPT)],
                                dinv_out.at[pl.ds(s * _RPT, _RPT)])

    return k(x2, idx4)


def _tc_tail(agg_ref, x_ref, x0_ref, dinv_ref, w_ref, g_ref, b_ref, y_ref):
    dsq = dinv_ref[...] * dinv_ref[...]
    agg = agg_ref[...] + x_ref[...] * dsq
    h = (1.0 - _ALPHA) * agg + _ALPHA * x0_ref[...]
    out = jnp.dot(h, w_ref[...], preferred_element_type=jnp.float32,
                  precision=lax.Precision.HIGHEST)
    out = jnp.maximum(out, 0.0)
    mean = jnp.sum(out, axis=0) / _N
    msq = jnp.sum(out * out, axis=0) / _N
    var = msq - mean * mean
    scale = g_ref[...] * lax.rsqrt(var + _EPS)[None, :]
    y_ref[...] = (out - mean[None, :]) * scale + b_ref[...]


def kernel(x, x_orig, edge_index, edge_weight, W, gamma, beta):
    row = edge_index[0]
    col = edge_index[1]
    pad = _EPAD - _E
    padidx = jnp.arange(pad, dtype=jnp.int32) % _N
    zpad = jnp.zeros((pad,), dtype=jnp.int32)
    rowp = jnp.concatenate([row, padidx]).reshape(_NW, _NBLK, _CB, _CH)
    colp = jnp.concatenate([col, padidx]).reshape(_NW, _NBLK, _CB, _CH)
    ewp = jnp.concatenate(
        [lax.bitcast_convert_type(edge_weight, jnp.int32), zpad]
    ).reshape(_NW, _NBLK, _CB, _CH)
    idx4 = jnp.stack([rowp, colp, ewp], axis=2)

    x2 = x.reshape(2 * _N, _DH)
    aggp, dinv = _sc_gcn_agg(x2, idx4)
    agg = aggp.transpose(1, 0, 2).reshape(_NPAD, _D)

    x_pad = jnp.pad(x, ((0, _NPAD - _N), (0, 0)))
    x0_pad = jnp.pad(x_orig, ((0, _NPAD - _N), (0, 0)))
    y_full = pl.pallas_call(
        _tc_tail,
        out_shape=jax.ShapeDtypeStruct((_NPAD, _D), jnp.float32),
    )(agg, x_pad, x0_pad, dinv[:, None], W, gamma[None, :], beta[None, :])

    y = y_full[:_N]
    return (y, x_orig, edge_index, edge_weight, x)


# speedup vs baseline: 1.2565x; 1.2417x over previous
import dataclasses
import functools

import jax
import jax.numpy as jnp
from jax import lax
from jax.experimental import pallas as pl
from jax.experimental.pallas import tpu as pltpu
from jax.experimental.pallas import tpu_sc as plsc

_N = 10000
_E = 320000
_D = 128
_ALPHA = 0.1
_EPS = 1e-5

_NC = 2
_NS = 16
_L = 16
_NW = _NC * _NS

_CH = 80
_CB = 8
_NBLK = 16
_NCHT = _CB * _NBLK
_EPAD = _NW * _NCHT * _CH
_NPAD = 10240
_RPT = _NPAD // _NS
_CSEG = 160


def _rsqrt16(d):
    i = plsc.bitcast(d, jnp.int32)
    i = jnp.int32(0x5F3759DF) - lax.shift_right_logical(i, 1)
    y = plsc.bitcast(i, jnp.float32)
    for _ in range(3):
        y = y * (1.5 - 0.5 * d * y * y)
    return y


def _sc_gcn_agg(x, idx4):
    mesh = plsc.VectorSubcoreMesh(core_axis_name="c", subcore_axis_name="s")
    cp = pltpu.CompilerParams()
    if "needs_layout_passes" in pltpu.CompilerParams.__dataclass_fields__:
        cp = dataclasses.replace(cp, needs_layout_passes=False)
    if "use_tc_tiling_on_sc" in pltpu.CompilerParams.__dataclass_fields__:
        cp = dataclasses.replace(cp, use_tc_tiling_on_sc=False)

    @functools.partial(
        pl.kernel,
        compiler_params=cp,
        out_type=(
            jax.ShapeDtypeStruct((_NC, _NPAD, _D), jnp.float32),
            jax.ShapeDtypeStruct((_NPAD,), jnp.float32),
        ),
        mesh=mesh,
        scratch_types=[
            pltpu.VMEM((3, _CB, _CH), jnp.int32),
            pltpu.VMEM((_NPAD,), jnp.float32),
            pltpu.VMEM((_CSEG, _L), jnp.float32),
            pltpu.VMEM((2, _CH, _L), jnp.float32),
            pltpu.VMEM((2, _CH, _D), jnp.float32),
            pltpu.VMEM((_CH,), jnp.float32),
            pltpu.VMEM_SHARED((_NPAD, _L), jnp.float32),
            pltpu.VMEM_SHARED((_NPAD, _D), jnp.float32),
            pltpu.VMEM_SHARED((_NPAD,), jnp.float32),
        ] + [pltpu.SemaphoreType.DMA] * 6,
    )
    def k(x_hbm, idx_hbm, agg_out, dinv_out,
          idxbuf, dinvv, cbuf, valbuf, msgbuf, normbuf,
          degmat, aggsh, dinvsh,
          sg0, sg1, ss0, ss1, sd0, sd1):
        semg = [sg0, sg1]
        sems = [ss0, ss1]
        semd = [sd0, sd1]
        c = lax.axis_index("c")
        s = lax.axis_index("s")
        iota16 = lax.iota(jnp.int32, _L)
        zero16i = jnp.zeros((_L,), jnp.int32)
        z16 = jnp.zeros((_L,), jnp.float32)

        with jax.named_scope("ph_zero"):
            @pl.loop(0, _CH)
            def _(e):
                valbuf[0, e, :] = z16
                valbuf[1, e, :] = z16
                for g in range(_D // _L):
                    msgbuf[0, e, pl.ds(g * _L, _L)] = z16

            for i in range(_RPT // _CH):
                base = s * _RPT + i * _CH
                pltpu.sync_copy(valbuf.at[0], degmat.at[pl.ds(base, _CH)])
                pltpu.sync_copy(msgbuf.at[0], aggsh.at[pl.ds(base, _CH)])

            plsc.subcore_barrier()

        with jax.named_scope("ph_deg"):
            for h in range(_NC):
                slab = h * _NS + s

                @pl.loop(0, _NBLK)
                def _(jb, slab=slab):
                    pltpu.sync_copy(idx_hbm.at[slab, jb], idxbuf)
                    hs = {}
                    for j8 in range(_CB):
                        b = j8 % 2
                        if j8 >= 2:
                            hs[b].wait()
                        for g in range(_CH // _L):
                            ew16 = plsc.bitcast(
                                idxbuf[2, j8, pl.ds(g * _L, _L)], jnp.float32)
                            plsc.store_scatter(
                                valbuf.at[b], [g * _L + iota16, zero16i], ew16)
                        hs[b] = pltpu.async_copy(
                            valbuf.at[b], degmat.at[idxbuf.at[1, j8]],
                            semd[b], add=True)
                    hs[0].wait()
                    hs[1].wait()

            plsc.subcore_barrier()

        with jax.named_scope("ph_rsqrt"):
            for seg in range(_RPT // _CSEG):
                base = s * _RPT + seg * _CSEG
                pltpu.sync_copy(degmat.at[pl.ds(base, _CSEG)], cbuf)

                @pl.loop(0, _CSEG // _L)
                def _(g, base=base):
                    r16 = g * _L + iota16
                    d16 = plsc.load_gather(cbuf, [r16, zero16i])
                    dinvv[pl.ds(base + g * _L, _L)] = _rsqrt16(d16 + 1.0)

            pltpu.sync_copy(dinvv.at[pl.ds(s * _RPT, _RPT)],
                            dinvsh.at[pl.ds(s * _RPT, _RPT)])
            plsc.subcore_barrier()
            pltpu.sync_copy(dinvsh, dinvv)

        with jax.named_scope("ph_agg"):
            aslab = c * _NS + s

            @pl.loop(0, _NBLK)
            def _(jb):
                pltpu.sync_copy(idx_hbm.at[aslab, jb], idxbuf)
                hg, hsc = {}, {}
                hg[0] = pltpu.async_copy(
                    x_hbm.at[idxbuf.at[0, 0]], msgbuf.at[0], semg[0])
                for j8 in range(_CB):
                    b = j8 % 2
                    hg[b].wait()
                    if j8 + 1 < _CB:
                        b1 = (j8 + 1) % 2
                        if j8 >= 1:
                            hsc[b1].wait()
                        hg[b1] = pltpu.async_copy(
                            x_hbm.at[idxbuf.at[0, j8 + 1]], msgbuf.at[b1],
                            semg[b1])
                    for g in range(_CH // _L):
                        sl = pl.ds(g * _L, _L)
                        r16 = idxbuf[0, j8, sl]
                        c16 = idxbuf[1, j8, sl]
                        ew16 = plsc.bitcast(idxbuf[2, j8, sl], jnp.float32)
                        dr = plsc.load_gather(dinvv, [r16])
                        dc = plsc.load_gather(dinvv, [c16])
                        normbuf[sl] = dr * ew16 * dc

                    @pl.loop(0, _CH // _L)
                    def _(eo, b=b):
                        n16 = normbuf[pl.ds(eo * _L, _L)]
                        for kk in range(_L):
                            ne = n16[kk]
                            for g in range(_D // _L):
                                sl = pl.ds(g * _L, _L)
                                msgbuf[b, eo * _L + kk, sl] = \
                                    msgbuf[b, eo * _L + kk, sl] * ne

                    hsc[b] = pltpu.async_copy(
                        msgbuf.at[b], aggsh.at[idxbuf.at[1, j8]],
                        sems[b], add=True)
                hsc[0].wait()
                hsc[1].wait()

            plsc.subcore_barrier()

        with jax.named_scope("ph_out"):
            pltpu.sync_copy(aggsh.at[pl.ds(s * _RPT, _RPT)],
                            agg_out.at[c, pl.ds(s * _RPT, _RPT)])

            @pl.when(c == 0)
            def _():
                pltpu.sync_copy(dinvv.at[pl.ds(s * _RPT, _RPT)],
                                dinv_out.at[pl.ds(s * _RPT, _RPT)])

    return k(x, idx4)


def _tc_tail(agg_ref, x_ref, x0_ref, dinv_ref, w_ref, g_ref, b_ref, y_ref):
    dsq = dinv_ref[...] * dinv_ref[...]
    agg = agg_ref[0] + agg_ref[1] + x_ref[...] * dsq
    h = (1.0 - _ALPHA) * agg + _ALPHA * x0_ref[...]
    out = jnp.dot(h, w_ref[...], preferred_element_type=jnp.float32,
                  precision=lax.Precision.HIGHEST)
    out = jnp.maximum(out, 0.0)
    mean = jnp.sum(out, axis=0) / _N
    msq = jnp.sum(out * out, axis=0) / _N
    var = msq - mean * mean
    scale = g_ref[...] * lax.rsqrt(var + _EPS)[None, :]
    y_ref[...] = (out - mean[None, :]) * scale + b_ref[...]


def kernel(x, x_orig, edge_index, edge_weight, W, gamma, beta):
    row = edge_index[0]
    col = edge_index[1]
    pad = _EPAD - _E
    padidx = jnp.arange(pad, dtype=jnp.int32) % _N
    zpad = jnp.zeros((pad,), dtype=jnp.int32)
    rowp = jnp.concatenate([row, padidx]).reshape(_NW, _NBLK, _CB, _CH)
    colp = jnp.concatenate([col, padidx]).reshape(_NW, _NBLK, _CB, _CH)
    ewp = jnp.concatenate(
        [lax.bitcast_convert_type(edge_weight, jnp.int32), zpad]
    ).reshape(_NW, _NBLK, _CB, _CH)
    idx4 = jnp.stack([rowp, colp, ewp], axis=2)

    aggp, dinv = _sc_gcn_agg(x, idx4)

    x_pad = jnp.pad(x, ((0, _NPAD - _N), (0, 0)))
    x0_pad = jnp.pad(x_orig, ((0, _NPAD - _N), (0, 0)))
    y_full = pl.pallas_call(
        _tc_tail,
        out_shape=jax.ShapeDtypeStruct((_NPAD, _D), jnp.float32),
    )(aggp, x_pad, x0_pad, dinv[:, None], W, gamma[None, :], beta[None, :])

    y = y_full[:_N]
    return (y, x_orig, edge_index, edge_weight, x)


# speedup vs baseline: 1.2751x; 1.0148x over previous
import dataclasses
import functools

import jax
import jax.numpy as jnp
from jax import lax
from jax.experimental import pallas as pl
from jax.experimental.pallas import tpu as pltpu
from jax.experimental.pallas import tpu_sc as plsc

_N = 10000
_E = 320000
_D = 128
_ALPHA = 0.1
_EPS = 1e-5

_NC = 2
_NS = 16
_L = 16
_NW = _NC * _NS

_CH = 80
_CB = 8
_NCH = _E // _CH
_NBT = 500
_NBLK = 16
_NPAD = 10240
_RPT = _NPAD // _NS
_CSEG = 160


def _rsqrt16(d):
    i = plsc.bitcast(d, jnp.int32)
    i = jnp.int32(0x5F3759DF) - lax.shift_right_logical(i, 1)
    y = plsc.bitcast(i, jnp.float32)
    for _ in range(3):
        y = y * (1.5 - 0.5 * d * y * y)
    return y


def _sc_gcn_agg(x, e3, ew3):
    mesh = plsc.VectorSubcoreMesh(core_axis_name="c", subcore_axis_name="s")
    cp = pltpu.CompilerParams()
    if "needs_layout_passes" in pltpu.CompilerParams.__dataclass_fields__:
        cp = dataclasses.replace(cp, needs_layout_passes=False)
    if "use_tc_tiling_on_sc" in pltpu.CompilerParams.__dataclass_fields__:
        cp = dataclasses.replace(cp, use_tc_tiling_on_sc=False)

    @functools.partial(
        pl.kernel,
        compiler_params=cp,
        out_type=(
            jax.ShapeDtypeStruct((_NC, _NPAD, _D), jnp.float32),
            jax.ShapeDtypeStruct((_NPAD,), jnp.float32),
        ),
        mesh=mesh,
        scratch_types=[
            pltpu.VMEM((_CB, _CH), jnp.int32),
            pltpu.VMEM((_CB, _CH), jnp.int32),
            pltpu.VMEM((_CB, _CH), jnp.float32),
            pltpu.VMEM((_NPAD,), jnp.float32),
            pltpu.VMEM((_CSEG, _L), jnp.float32),
            pltpu.VMEM((2, _CH, _L), jnp.float32),
            pltpu.VMEM((2, _CH, _D), jnp.float32),
            pltpu.VMEM((_CH,), jnp.float32),
            pltpu.VMEM_SHARED((_NPAD, _L), jnp.float32),
            pltpu.VMEM_SHARED((_NPAD, _D), jnp.float32),
            pltpu.VMEM_SHARED((_NPAD,), jnp.float32),
        ] + [pltpu.SemaphoreType.DMA] * 6,
    )
    def k(x_hbm, e3_hbm, ew3_hbm, agg_out, dinv_out,
          rowblk, colblk, ewblk, dinvv, cbuf, valbuf, msgbuf, normbuf,
          degmat, aggsh, dinvsh,
          sg0, sg1, ss0, ss1, sd0, sd1):
        semg = [sg0, sg1]
        sems = [ss0, ss1]
        semd = [sd0, sd1]
        c = lax.axis_index("c")
        s = lax.axis_index("s")
        iota16 = lax.iota(jnp.int32, _L)
        zero16i = jnp.zeros((_L,), jnp.int32)
        z16 = jnp.zeros((_L,), jnp.float32)

        with jax.named_scope("ph_zero"):
            @pl.loop(0, _CH)
            def _(e):
                valbuf[0, e, :] = z16
                valbuf[1, e, :] = z16
                for g in range(_D // _L):
                    msgbuf[0, e, pl.ds(g * _L, _L)] = z16

            for i in range(_RPT // _CH):
                base = s * _RPT + i * _CH
                pltpu.sync_copy(valbuf.at[0], degmat.at[pl.ds(base, _CH)])
                pltpu.sync_copy(msgbuf.at[0], aggsh.at[pl.ds(base, _CH)])

            plsc.subcore_barrier()

        with jax.named_scope("ph_deg"):
            for h in range(_NC):
                slab = h * _NS + s

                @pl.loop(0, _NBLK)
                def _(jb, slab=slab):
                    gb = slab * _NBLK + jb

                    @pl.when(gb < _NBT)
                    def _():
                        pltpu.sync_copy(e3_hbm.at[1, pl.ds(gb * _CB, _CB)],
                                        colblk)
                        pltpu.sync_copy(ew3_hbm.at[pl.ds(gb * _CB, _CB)],
                                        ewblk)
                        hs = {}
                        for j8 in range(_CB):
                            b = j8 % 2
                            if j8 >= 2:
                                hs[b].wait()
                            for g in range(_CH // _L):
                                ew16 = ewblk[j8, pl.ds(g * _L, _L)]
                                plsc.store_scatter(
                                    valbuf.at[b], [g * _L + iota16, zero16i],
                                    ew16)
                            hs[b] = pltpu.async_copy(
                                valbuf.at[b], degmat.at[colblk.at[j8]],
                                semd[b], add=True)
                        hs[0].wait()
                        hs[1].wait()

            plsc.subcore_barrier()

        with jax.named_scope("ph_rsqrt"):
            for seg in range(_RPT // _CSEG):
                base = s * _RPT + seg * _CSEG
                pltpu.sync_copy(degmat.at[pl.ds(base, _CSEG)], cbuf)

                @pl.loop(0, _CSEG // _L)
                def _(g, base=base):
                    r16 = g * _L + iota16
                    d16 = plsc.load_gather(cbuf, [r16, zero16i])
                    dinvv[pl.ds(base + g * _L, _L)] = _rsqrt16(d16 + 1.0)

            pltpu.sync_copy(dinvv.at[pl.ds(s * _RPT, _RPT)],
                            dinvsh.at[pl.ds(s * _RPT, _RPT)])
            plsc.subcore_barrier()
            pltpu.sync_copy(dinvsh, dinvv)

        with jax.named_scope("ph_agg"):
            aslab = c * _NS + s

            @pl.loop(0, _NBLK)
            def _(jb):
                gb = aslab * _NBLK + jb

                @pl.when(gb < _NBT)
                def _():
                    pltpu.sync_copy(e3_hbm.at[0, pl.ds(gb * _CB, _CB)],
                                    rowblk)
                    pltpu.sync_copy(e3_hbm.at[1, pl.ds(gb * _CB, _CB)],
                                    colblk)
                    pltpu.sync_copy(ew3_hbm.at[pl.ds(gb * _CB, _CB)], ewblk)
                    hg, hsc = {}, {}
                    hg[0] = pltpu.async_copy(
                        x_hbm.at[rowblk.at[0]], msgbuf.at[0], semg[0])
                    for j8 in range(_CB):
                        b = j8 % 2
                        hg[b].wait()
                        if j8 + 1 < _CB:
                            b1 = (j8 + 1) % 2
                            if j8 >= 1:
                                hsc[b1].wait()
                            hg[b1] = pltpu.async_copy(
                                x_hbm.at[rowblk.at[j8 + 1]], msgbuf.at[b1],
                                semg[b1])
                        for g in range(_CH // _L):
                            sl = pl.ds(g * _L, _L)
                            r16 = rowblk[j8, sl]
                            c16 = colblk[j8, sl]
                            ew16 = ewblk[j8, sl]
                            dr = plsc.load_gather(dinvv, [r16])
                            dc = plsc.load_gather(dinvv, [c16])
                            normbuf[sl] = dr * ew16 * dc

                        @pl.loop(0, _CH // _L)
                        def _(eo, b=b):
                            n16 = normbuf[pl.ds(eo * _L, _L)]
                            for kk in range(_L):
                                ne = n16[kk]
                                for g in range(_D // _L):
                                    sl = pl.ds(g * _L, _L)
                                    msgbuf[b, eo * _L + kk, sl] = \
                                        msgbuf[b, eo * _L + kk, sl] * ne

                        hsc[b] = pltpu.async_copy(
                            msgbuf.at[b], aggsh.at[colblk.at[j8]],
                            sems[b], add=True)
                    hsc[0].wait()
                    hsc[1].wait()

            plsc.subcore_barrier()

        with jax.named_scope("ph_out"):
            pltpu.sync_copy(aggsh.at[pl.ds(s * _RPT, _RPT)],
                            agg_out.at[c, pl.ds(s * _RPT, _RPT)])

            @pl.when(c == 0)
            def _():
                pltpu.sync_copy(dinvv.at[pl.ds(s * _RPT, _RPT)],
                                dinv_out.at[pl.ds(s * _RPT, _RPT)])

    return k(x, e3, ew3)


def _tc_tail(agg_ref, x_ref, x0_ref, dinv_ref, w_ref, g_ref, b_ref, y_ref):
    dsq = dinv_ref[...] * dinv_ref[...]
    agg = agg_ref[0, :_N, :] + agg_ref[1, :_N, :] + x_ref[...] * dsq
    h = (1.0 - _ALPHA) * agg + _ALPHA * x0_ref[...]
    out = jnp.dot(h, w_ref[...], preferred_element_type=jnp.float32,
                  precision=lax.Precision.HIGHEST)
    out = jnp.maximum(out, 0.0)
    mean = jnp.sum(out, axis=0) / _N
    msq = jnp.sum(out * out, axis=0) / _N
    var = msq - mean * mean
    scale = g_ref[...] * lax.rsqrt(var + _EPS)[None, :]
    y_ref[...] = (out - mean[None, :]) * scale + b_ref[...]


def kernel(x, x_orig, edge_index, edge_weight, W, gamma, beta):
    e3 = edge_index.reshape(2, _NCH, _CH)
    ew3 = edge_weight.reshape(_NCH, _CH)

    aggp, dinv = _sc_gcn_agg(x, e3, ew3)

    y = pl.pallas_call(
        _tc_tail,
        out_shape=jax.ShapeDtypeStruct((_N, _D), jnp.float32),
    )(aggp, x, x_orig, dinv[:_N, None], W, gamma[None, :], beta[None, :])

    return (y, x_orig, edge_index, edge_weight, x)
